# baseline (speedup 1.0000x reference)
import jax
import jax.numpy as jnp
from jax.experimental import pallas as pl
from jax.experimental.pallas import tpu as pltpu

_N = 10000
_D = 128
_HID = 32
_PRE = 12
_BLOCK = 5000


def _fused_body(x_ref, wz_ref, bz_ref, wh_ref, bh_ref, wl_ref, bl_ref, o_ref):
    az = (wz_ref[0, 0, :_D, :] + wz_ref[1, 0, :_D, :])
    ah = (wh_ref[0, 0, :_D, :] + wh_ref[1, 0, :_D, :])
    xb = x_ref[:]
    z = jax.nn.sigmoid(jnp.dot(xb, az, preferred_element_type=jnp.float32)
                       + bz_ref[:])
    t = jnp.tanh(jnp.dot(xb, ah, preferred_element_type=jnp.float32)
                 + bh_ref[:])
    h = jnp.maximum((1.0 - z) * t, 0.0)
    o_ref[:] = jnp.dot(h, wl_ref[:], preferred_element_type=jnp.float32) + bl_ref[:]


def kernel(x, edge_index, edge_weight, Wz, bz, Wr, br, Wh, bh, Wl, bl):
    del edge_index, edge_weight, Wr, br
    grid = (_N // _BLOCK,)
    full = lambda *shape: pl.BlockSpec(shape, lambda i: (0,) * len(shape))
    return pl.pallas_call(
        _fused_body,
        grid=grid,
        in_specs=[
            pl.BlockSpec((_BLOCK, _D), lambda i: (i, 0)),
            full(2, 1, _D + _HID, _HID),
            full(1, _HID),
            full(2, 1, _D + _HID, _HID),
            full(1, _HID),
            full(_HID, _PRE),
            full(1, _PRE),
        ],
        out_specs=pl.BlockSpec((_BLOCK, _PRE), lambda i: (i, 0)),
        out_shape=jax.ShapeDtypeStruct((_N, _PRE), jnp.float32),
        compiler_params=pltpu.CompilerParams(
            dimension_semantics=("parallel",)),
    )(x, Wz, bz.reshape(1, _HID), Wh, bh.reshape(1, _HID), Wl,
      bl.reshape(1, _PRE))


# speedup vs baseline: 1.0578x; 1.0578x over previous
import jax
import jax.numpy as jnp
from jax.experimental import pallas as pl

_N = 10000
_D = 128
_HID = 32
_PRE = 12
_BLOCK = 10000


def _fused_body(x_ref, wz_ref, bz_ref, wh_ref, bh_ref, wl_ref, bl_ref, o_ref):
    az = (wz_ref[0, 0, :_D, :] + wz_ref[1, 0, :_D, :]) * 0.5
    ah = (wh_ref[0, 0, :_D, :] + wh_ref[1, 0, :_D, :])
    xb = x_ref[:]
    vz = jnp.tanh(jnp.dot(xb, az, preferred_element_type=jnp.float32)
                  + bz_ref[:] * 0.5)
    t = jnp.tanh(jnp.dot(xb, ah, preferred_element_type=jnp.float32)
                 + bh_ref[:])
    h = jnp.maximum((0.5 - 0.5 * vz) * t, 0.0)
    o_ref[:] = jnp.dot(h, wl_ref[:], preferred_element_type=jnp.float32) + bl_ref[:]


def kernel(x, edge_index, edge_weight, Wz, bz, Wr, br, Wh, bh, Wl, bl):
    del edge_index, edge_weight, Wr, br
    grid = (_N // _BLOCK,)
    full = lambda *shape: pl.BlockSpec(shape, lambda i: (0,) * len(shape))
    return pl.pallas_call(
        _fused_body,
        grid=grid,
        in_specs=[
            pl.BlockSpec((_BLOCK, _D), lambda i: (i, 0)),
            full(2, 1, _D + _HID, _HID),
            full(1, _HID),
            full(2, 1, _D + _HID, _HID),
            full(1, _HID),
            full(_HID, _PRE),
            full(1, _PRE),
        ],
        out_specs=pl.BlockSpec((_BLOCK, _PRE), lambda i: (i, 0)),
        out_shape=jax.ShapeDtypeStruct((_N, _PRE), jnp.float32),
    )(x, Wz, bz.reshape(1, _HID), Wh, bh.reshape(1, _HID), Wl,
      bl.reshape(1, _PRE))


# speedup vs baseline: 1.0634x; 1.0053x over previous
import jax
import jax.numpy as jnp
from jax.experimental import pallas as pl

_N = 10000
_D = 128
_HID = 32
_PRE = 12
_P = 5
_ROWS = _N // _P


def _fused_body(x0, x1, x2, x3, x4, wz_ref, bz_ref, wh_ref, bh_ref, wl_ref,
                bl_ref, o_ref):
    az = (wz_ref[0, 0, :_D, :] + wz_ref[1, 0, :_D, :]) * 0.5
    ah = wh_ref[0, 0, :_D, :] + wh_ref[1, 0, :_D, :]
    bz2 = bz_ref[:] * 0.5
    wl = wl_ref[:]
    for k, x_ref in enumerate((x0, x1, x2, x3, x4)):
        xb = x_ref[:]
        vz = jnp.tanh(jnp.dot(xb, az, preferred_element_type=jnp.float32)
                      + bz2)
        t = jnp.tanh(jnp.dot(xb, ah, preferred_element_type=jnp.float32)
                     + bh_ref[:])
        h = jnp.maximum((0.5 - 0.5 * vz) * t, 0.0)
        o_ref[pl.ds(k * _ROWS, _ROWS), :] = (
            jnp.dot(h, wl, preferred_element_type=jnp.float32) + bl_ref[:])


def kernel(x, edge_index, edge_weight, Wz, bz, Wr, br, Wh, bh, Wl, bl):
    del edge_index, edge_weight, Wr, br
    full = lambda *shape: pl.BlockSpec(shape, lambda i: (0,) * len(shape))
    part = lambda k: pl.BlockSpec((_ROWS, _D), lambda i, k=k: (k, 0))
    return pl.pallas_call(
        _fused_body,
        grid=(1,),
        in_specs=[
            part(0), part(1), part(2), part(3), part(4),
            full(2, 1, _D + _HID, _HID),
            full(1, _HID),
            full(2, 1, _D + _HID, _HID),
            full(1, _HID),
            full(_HID, _PRE),
            full(1, _PRE),
        ],
        out_specs=full(_N, _PRE),
        out_shape=jax.ShapeDtypeStruct((_N, _PRE), jnp.float32),
    )(x, x, x, x, x, Wz, bz.reshape(1, _HID), Wh, bh.reshape(1, _HID), Wl,
      bl.reshape(1, _PRE))
